# Initial kernel scaffold; baseline (speedup 1.0000x reference)
import jax
import jax.numpy as jnp
from jax.experimental import pallas as pl


def kernel(labels, label_embed_table):
    raise NotImplementedError("write your pallas kernel here")


# speedup vs baseline: 1.7349x; 1.7349x over previous
import functools

import jax
import jax.numpy as jnp
from jax import lax
from jax.experimental import pallas as pl
from jax.experimental.pallas import tpu as pltpu
from jax.experimental.pallas import tpu_sc as plsc

NUM_CLASSES = 80
D = 256
NC, NS = 2, 16
NW = NC * NS
B_TOT = 1024 * 300
B_PER_W = B_TOT // NW
CHUNK = 120
NBUF = 2
NCHUNK = B_PER_W // CHUNK


def _worker(table_hbm, idx_hbm, out_hbm, idx_v, buf, sem0, sem1):
    sems = (sem0, sem1)
    wid = lax.axis_index("s") * NC + lax.axis_index("c")
    base = pl.multiple_of(wid * B_PER_W, 8)
    pltpu.sync_copy(idx_hbm.at[pl.ds(base, B_PER_W)], idx_v)

    def start(i, b):
        off = pl.multiple_of(i * CHUNK, 8)
        pltpu.async_copy(
            table_hbm.at[idx_v.at[pl.ds(off, CHUNK)]], buf.at[b], sems[b])

    def finish(i, b):
        pltpu.make_async_copy(
            table_hbm.at[idx_v.at[pl.ds(0, CHUNK)]], buf.at[b], sems[b]).wait()
        pltpu.sync_copy(buf.at[b], out_hbm.at[pl.ds(base + i * CHUNK, CHUNK)])

    for b in range(NBUF):
        start(b, b)

    def outer(g, carry):
        for b in range(NBUF):
            i = NBUF * g + b
            finish(i, b)
            start(i + NBUF, b)
        return carry

    lax.fori_loop(0, NCHUNK // NBUF - 1, outer, 0)
    for b in range(NBUF):
        finish(NCHUNK - NBUF + b, b)


_sc_gather = functools.partial(
    pl.kernel,
    out_type=jax.ShapeDtypeStruct((B_TOT, D), jnp.float32),
    mesh=plsc.VectorSubcoreMesh(core_axis_name="c", subcore_axis_name="s"),
    scratch_types=[
        pltpu.VMEM((B_PER_W,), jnp.int32),
        pltpu.VMEM((NBUF, CHUNK, D), jnp.float32),
        pltpu.SemaphoreType.DMA,
        pltpu.SemaphoreType.DMA,
    ],
)(_worker)


def kernel(labels, label_embed_table):
    bsz, n = labels.shape
    ones = jnp.ones((label_embed_table.shape[0], 1), label_embed_table.dtype)
    table = jnp.concatenate([label_embed_table, ones], axis=-1)
    out = _sc_gather(table, labels.reshape(-1))
    return out.reshape(bsz, n, D)


# speedup vs baseline: 1.7450x; 1.0058x over previous
import functools

import jax
import jax.numpy as jnp
from jax import lax
from jax.experimental import pallas as pl
from jax.experimental.pallas import tpu as pltpu
from jax.experimental.pallas import tpu_sc as plsc

NUM_CLASSES = 80
D = 256
NC, NS = 2, 16
NW = NC * NS
B_TOT = 1024 * 300
B_PER_W = B_TOT // NW
CHUNK = 128
NBUF = 3
NCHUNK = B_PER_W // CHUNK


def _worker(table_hbm, idx_hbm, out_hbm, idx_v, buf, sem0, sem1, sem2):
    sems = (sem0, sem1, sem2)
    wid = lax.axis_index("s") * NC + lax.axis_index("c")
    base = pl.multiple_of(wid * B_PER_W, 8)
    pltpu.sync_copy(idx_hbm.at[pl.ds(base, B_PER_W)], idx_v)

    def start(i, b):
        off = pl.multiple_of(i * CHUNK, 8)
        pltpu.async_copy(
            table_hbm.at[idx_v.at[pl.ds(off, CHUNK)]], buf.at[b], sems[b])

    def finish(i, b):
        pltpu.make_async_copy(
            table_hbm.at[idx_v.at[pl.ds(0, CHUNK)]], buf.at[b], sems[b]).wait()
        pltpu.sync_copy(buf.at[b], out_hbm.at[pl.ds(base + i * CHUNK, CHUNK)])

    for b in range(NBUF):
        start(b, b)

    def outer(g, carry):
        for b in range(NBUF):
            i = NBUF * g + b
            finish(i, b)
            start(i + NBUF, b)
        return carry

    lax.fori_loop(0, NCHUNK // NBUF - 1, outer, 0)
    for b in range(NBUF):
        finish(NCHUNK - NBUF + b, b)


_sc_gather = functools.partial(
    pl.kernel,
    out_type=jax.ShapeDtypeStruct((B_TOT, D), jnp.float32),
    mesh=plsc.VectorSubcoreMesh(core_axis_name="c", subcore_axis_name="s"),
    scratch_types=[
        pltpu.VMEM((B_PER_W,), jnp.int32),
        pltpu.VMEM((NBUF, CHUNK, D), jnp.float32),
        pltpu.SemaphoreType.DMA,
        pltpu.SemaphoreType.DMA,
        pltpu.SemaphoreType.DMA,
    ],
)(_worker)


def kernel(labels, label_embed_table):
    bsz, n = labels.shape
    ones = jnp.ones((label_embed_table.shape[0], 1), label_embed_table.dtype)
    table = jnp.concatenate([label_embed_table, ones], axis=-1)
    out = _sc_gather(table, labels.reshape(-1))
    return out.reshape(bsz, n, D)


# speedup vs baseline: 2.2568x; 1.2933x over previous
import functools

import jax
import jax.numpy as jnp
from jax import lax
from jax.experimental import pallas as pl
from jax.experimental.pallas import tpu as pltpu
from jax.experimental.pallas import tpu_sc as plsc

NUM_CLASSES = 80
D = 256
NC, NS = 2, 16
NW = NC * NS
BSZ, N = 1024, 300
NPAD = 304
B_PER_W = BSZ // NW
OFFS = (0, 104, 200)
SIZES = (104, 96, 104)
NBUF = 3


def _worker(table_hbm, idx_hbm, out_hbm, idx_v, buf, sem0, sem1, sem2):
    sems = (sem0, sem1, sem2)
    wid = lax.axis_index("s") * NC + lax.axis_index("c")
    b0 = wid * B_PER_W
    base = pl.multiple_of(b0 * NPAD, 8)
    pltpu.sync_copy(idx_hbm.at[pl.ds(base, B_PER_W * NPAD)], idx_v)

    def start(g, j):
        off = pl.multiple_of(g * NPAD + OFFS[j], 8)
        pltpu.async_copy(
            table_hbm.at[idx_v.at[pl.ds(off, SIZES[j])]],
            buf.at[j, pl.ds(0, SIZES[j])], sems[j])

    def finish(g, j):
        pltpu.make_async_copy(
            table_hbm.at[idx_v.at[pl.ds(0, SIZES[j])]],
            buf.at[j, pl.ds(0, SIZES[j])], sems[j]).wait()
        pltpu.sync_copy(buf.at[j, pl.ds(0, SIZES[j])],
                        out_hbm.at[b0 + g, pl.ds(OFFS[j], SIZES[j])])

    for j in range(NBUF):
        start(0, j)

    def outer(g, carry):
        for j in range(NBUF):
            finish(g, j)
            start(g + 1, j)
        return carry

    lax.fori_loop(0, B_PER_W - 1, outer, 0)
    for j in range(NBUF):
        finish(B_PER_W - 1, j)


_sc_gather = functools.partial(
    pl.kernel,
    out_type=jax.ShapeDtypeStruct((BSZ, NPAD, D), jnp.float32),
    mesh=plsc.VectorSubcoreMesh(core_axis_name="c", subcore_axis_name="s"),
    scratch_types=[
        pltpu.VMEM((B_PER_W * NPAD,), jnp.int32),
        pltpu.VMEM((NBUF, max(SIZES), D), jnp.float32),
        pltpu.SemaphoreType.DMA,
        pltpu.SemaphoreType.DMA,
        pltpu.SemaphoreType.DMA,
    ],
)(_worker)


def kernel(labels, label_embed_table):
    ones = jnp.ones((label_embed_table.shape[0], 1), label_embed_table.dtype)
    table = jnp.concatenate([label_embed_table, ones], axis=-1)
    labels_p = jnp.pad(labels, ((0, 0), (0, NPAD - N))).reshape(-1)
    return _sc_gather(table, labels_p)[:, :N, :]


# speedup vs baseline: 2.2706x; 1.0061x over previous
import functools

import jax
import jax.numpy as jnp
from jax import lax
from jax.experimental import pallas as pl
from jax.experimental.pallas import tpu as pltpu
from jax.experimental.pallas import tpu_sc as plsc

NUM_CLASSES = 80
D = 256
NC, NS = 2, 16
NW = NC * NS
BSZ, N = 1024, 300
NPAD = 304
B_PER_W = BSZ // NW
OFFS = (0, 56, 104, 152, 200, 248)
SIZES = (56, 48, 48, 48, 48, 56)
NSLOT = 6


def _worker(table_hbm, idx_hbm, out_hbm, idx_v, buf, gsems, wsems):
    wid = lax.axis_index("s") * NC + lax.axis_index("c")
    b0 = wid * B_PER_W
    base = pl.multiple_of(b0 * NPAD, 8)
    pltpu.sync_copy(idx_hbm.at[pl.ds(base, B_PER_W * NPAD)], idx_v)

    def start(g, j):
        off = pl.multiple_of(g * NPAD + OFFS[j], 8)
        pltpu.async_copy(
            table_hbm.at[idx_v.at[pl.ds(off, SIZES[j])]],
            buf.at[j, pl.ds(0, SIZES[j])], gsems[j])

    def wait_gather(j):
        pltpu.make_async_copy(
            table_hbm.at[idx_v.at[pl.ds(0, SIZES[j])]],
            buf.at[j, pl.ds(0, SIZES[j])], gsems[j]).wait()

    def write(g, j):
        pltpu.async_copy(
            buf.at[j, pl.ds(0, SIZES[j])],
            out_hbm.at[b0 + g, pl.ds(OFFS[j], SIZES[j])], wsems[j])

    def wait_write(g, j):
        pltpu.make_async_copy(
            buf.at[j, pl.ds(0, SIZES[j])],
            out_hbm.at[b0 + g, pl.ds(OFFS[j], SIZES[j])], wsems[j]).wait()

    for j in range(3):
        start(0, j)

    for j in range(NSLOT):
        wait_gather(j)
        write(0, j)
        if j < 3:
            start(0, j + 3)
        else:
            wait_write(0, j - 3)
            start(1, j - 3)

    def outer(g, carry):
        for j in range(NSLOT):
            wait_gather(j)
            write(g, j)
            if j < 3:
                wait_write(g - 1, j + 3)
                start(g, j + 3)
            else:
                wait_write(g, j - 3)
                start(g + 1, j - 3)
        return carry

    lax.fori_loop(1, B_PER_W - 1, outer, 0)

    g_last = B_PER_W - 1
    for j in range(NSLOT):
        wait_gather(j)
        write(g_last, j)
        if j < 3:
            wait_write(g_last - 1, j + 3)
            start(g_last, j + 3)
    for j in range(NSLOT):
        wait_write(g_last, j)


_sc_gather = functools.partial(
    pl.kernel,
    out_type=jax.ShapeDtypeStruct((BSZ, NPAD, D), jnp.float32),
    mesh=plsc.VectorSubcoreMesh(core_axis_name="c", subcore_axis_name="s"),
    scratch_types=[
        pltpu.VMEM((B_PER_W * NPAD,), jnp.int32),
        pltpu.VMEM((NSLOT, max(SIZES), D), jnp.float32),
        [pltpu.SemaphoreType.DMA] * NSLOT,
        [pltpu.SemaphoreType.DMA] * NSLOT,
    ],
)(_worker)


def kernel(labels, label_embed_table):
    ones = jnp.ones((label_embed_table.shape[0], 1), label_embed_table.dtype)
    table = jnp.concatenate([label_embed_table, ones], axis=-1)
    labels_p = jnp.pad(labels, ((0, 0), (0, NPAD - N))).reshape(-1)
    return _sc_gather(table, labels_p)[:, :N, :]


# speedup vs baseline: 4.2310x; 1.8634x over previous
import functools

import jax
import jax.numpy as jnp
from jax import lax
from jax.experimental import pallas as pl
from jax.experimental.pallas import tpu as pltpu
from jax.experimental.pallas import tpu_sc as plsc

NUM_CLASSES = 80
D = 256
NC, NS = 2, 16
NW = NC * NS
BSZ, N = 1024, 300
NPAD = 304
B_PER_W = BSZ // NW
OFFS = (0, 56, 104, 152, 200, 248)
SIZES = (56, 48, 48, 48, 48, 56)
NSLOT = 6


def _worker(table_hbm, idx_hbm, out_hbm, idx_v, buf, gsems, wsems):
    wid = lax.axis_index("s") * NC + lax.axis_index("c")
    b0 = wid * B_PER_W
    base = pl.multiple_of(b0 * NPAD, 8)
    pltpu.sync_copy(idx_hbm.at[pl.ds(base, B_PER_W * NPAD)], idx_v)

    def start(g, j):
        off = pl.multiple_of(g * NPAD + OFFS[j], 8)
        pltpu.async_copy(
            table_hbm.at[idx_v.at[pl.ds(off, SIZES[j])]],
            buf.at[j, pl.ds(0, SIZES[j])], gsems[j])

    def wait_gather(j):
        pltpu.make_async_copy(
            table_hbm.at[idx_v.at[pl.ds(0, SIZES[j])]],
            buf.at[j, pl.ds(0, SIZES[j])], gsems[j]).wait()

    def write(g, j):
        pltpu.async_copy(
            buf.at[j, pl.ds(0, SIZES[j])],
            out_hbm.at[b0 + g, pl.ds(OFFS[j], SIZES[j])], wsems[j])

    def wait_write(g, j):
        pltpu.make_async_copy(
            buf.at[j, pl.ds(0, SIZES[j])],
            out_hbm.at[b0 + g, pl.ds(OFFS[j], SIZES[j])], wsems[j]).wait()

    for j in range(3):
        start(0, j)

    for j in range(NSLOT):
        wait_gather(j)
        write(0, j)
        if j < 3:
            start(0, j + 3)
        else:
            wait_write(0, j - 3)
            start(1, j - 3)

    def outer(g, carry):
        for j in range(NSLOT):
            wait_gather(j)
            write(g, j)
            if j < 3:
                wait_write(g - 1, j + 3)
                start(g, j + 3)
            else:
                wait_write(g, j - 3)
                start(g + 1, j - 3)
        return carry

    lax.fori_loop(1, B_PER_W - 1, outer, 0)

    g_last = B_PER_W - 1
    for j in range(NSLOT):
        wait_gather(j)
        write(g_last, j)
        if j < 3:
            wait_write(g_last - 1, j + 3)
            start(g_last, j + 3)
    for j in range(NSLOT):
        wait_write(g_last, j)


_sc_gather = functools.partial(
    pl.kernel,
    out_type=jax.ShapeDtypeStruct((BSZ, NPAD, D), jnp.float32),
    mesh=plsc.VectorSubcoreMesh(core_axis_name="c", subcore_axis_name="s"),
    scratch_types=[
        pltpu.VMEM((B_PER_W * NPAD,), jnp.int32),
        pltpu.VMEM((NSLOT, max(SIZES), D), jnp.float32),
        [pltpu.SemaphoreType.DMA] * NSLOT,
        [pltpu.SemaphoreType.DMA] * NSLOT,
    ],
)(_worker)


NREP = 64


def kernel(labels, label_embed_table):
    ones = jnp.ones((label_embed_table.shape[0], 1), label_embed_table.dtype)
    table = jnp.concatenate([label_embed_table, ones], axis=-1)
    table_rep = jnp.tile(table, (NREP, 1))
    labels_p = jnp.pad(labels, ((0, 0), (0, NPAD - N))).reshape(-1)
    nrows = label_embed_table.shape[0]
    rot = (jnp.arange(labels_p.shape[0], dtype=jnp.int32) % NREP) * nrows
    return _sc_gather(table_rep, labels_p + rot)[:, :N, :]
